# baseline (speedup 1.0000x reference)
import functools

import jax
import jax.numpy as jnp
from jax import lax
from jax.experimental import pallas as pl
from jax.experimental.pallas import tpu as pltpu
from jax.experimental.pallas import tpu_sc as plsc

_NC = 2
_NS = 16
_NW = _NC * _NS
_BLK = 128
_PAIRS = 2
_CHUNK = _PAIRS * _BLK
_NBUF = 2
_LPAD = _BLK + 1
_FCOLS = 256


@functools.cache
def _build_table_format(vocab, dim):
    n_chunks = vocab // _FCOLS
    rows_out = vocab // 2
    rows_chunk = _FCOLS // 2
    pitch = 2 * dim + 1

    mesh = plsc.VectorSubcoreMesh(core_axis_name="c", subcore_axis_name="s")

    @functools.partial(
        pl.kernel,
        out_type=jax.ShapeDtypeStruct((rows_out, 2 * dim), jnp.float32),
        mesh=mesh,
        compiler_params=pltpu.CompilerParams(
            use_tc_tiling_on_sc=True, needs_layout_passes=False),
        scratch_types=[
            pltpu.VMEM((_NBUF, dim, _FCOLS), jnp.float32),
            pltpu.VMEM((_NBUF, rows_chunk, pitch), jnp.float32),
            pltpu.SemaphoreType.DMA,
            pltpu.SemaphoreType.DMA,
            pltpu.SemaphoreType.DMA,
            pltpu.SemaphoreType.DMA,
        ],
    )
    def fmt_kernel(tab_hbm, tail_hbm, out_hbm, src_v, dst_v, isem0, isem1,
                   osem0, osem1):
        c = lax.axis_index("c")
        s_ax = lax.axis_index("s")
        wid = s_ax * _NC + c
        isems = (isem0, isem1)
        osems = (osem0, osem1)
        iota16 = jax.lax.iota(jnp.int32, 16)
        row_base = iota16 // 2
        col_base = (iota16 % 2) * dim

        def chunk_id(g):
            return g * _NW + wid

        def in_copies(g, b, fire):
            desc = pltpu.make_async_copy(
                tab_hbm.at[:, pl.ds(chunk_id(g) * _FCOLS, _FCOLS)],
                src_v.at[b], isems[b])
            desc.start() if fire else desc.wait()

        def out_copies(g, b, fire):
            desc = pltpu.make_async_copy(
                dst_v.at[b, :, pl.ds(0, 2 * dim)],
                out_hbm.at[pl.ds(chunk_id(g) * rows_chunk, rows_chunk)],
                osems[b])
            desc.start() if fire else desc.wait()

        def transpose_chunk(b):
            def t_body(k, carry):
                for dd in range(2):
                    d = k * 2 + dd
                    col_vec = col_base + d
                    for u16 in range(_FCOLS // 16):
                        vals = src_v[b, d, pl.ds(u16 * 16, 16)]
                        plsc.store_scatter(
                            dst_v.at[b],
                            [row_base + (u16 * 8), col_vec], vals)
                return carry

            lax.fori_loop(0, dim // 2, t_body, 0)

        def guarded(g, fn_, *a, **kw):
            @pl.when(chunk_id(g) < n_chunks)
            def _():
                fn_(*a, **kw)

        for b in range(_NBUF):
            guarded(b, in_copies, b, b, fire=True)

        per_worker = (n_chunks + _NW - 1) // _NW
        per_worker += per_worker % _NBUF

        def loop_body(t, carry):
            for b in range(_NBUF):
                g = t * _NBUF + b
                guarded(g, in_copies, g, b, fire=False)

                @pl.when((g >= _NBUF) & (chunk_id(g - _NBUF) < n_chunks))
                def _():
                    out_copies(g - _NBUF, b, fire=False)

                guarded(g, transpose_chunk, b)
                guarded(g, out_copies, g, b, fire=True)

                @pl.when(chunk_id(g + _NBUF) < n_chunks)
                def _():
                    in_copies(g + _NBUF, b, fire=True)
            return carry

        lax.fori_loop(0, per_worker // _NBUF, loop_body, 0)

        for b in range(_NBUF):
            g = per_worker - _NBUF + b
            guarded(g, out_copies, g, b, fire=False)

        tail = vocab % _FCOLS
        if tail:

            @pl.when(wid == 0)
            def _():
                pltpu.sync_copy(
                    tail_hbm,
                    out_hbm.at[pl.ds(n_chunks * rows_chunk, tail // 2)])

    return fmt_kernel


@functools.cache
def _build(batch, seq, vocab, dim):
    n_bblk = batch // _BLK
    n_pairs = seq * n_bblk
    pairs_per_worker = n_pairs // _NW
    chunks_per_worker = pairs_per_worker // _PAIRS
    assert chunks_per_worker % _NBUF == 0
    idx_per_worker = pairs_per_worker * _BLK
    n_dblk = dim // 8
    tile_rows = _PAIRS * 8

    mesh = plsc.VectorSubcoreMesh(core_axis_name="c", subcore_axis_name="s")

    @functools.partial(
        pl.kernel,
        out_type=jax.ShapeDtypeStruct((seq, n_dblk * n_bblk * 8, _BLK),
                                      jnp.float32),
        mesh=mesh,
        compiler_params=pltpu.CompilerParams(
            use_tc_tiling_on_sc=False, needs_layout_passes=False),
        scratch_types=[
            pltpu.VMEM((idx_per_worker,), jnp.int32),
            pltpu.VMEM((_NBUF, _CHUNK, dim), jnp.float32),
            pltpu.VMEM((_NBUF, n_dblk, tile_rows, _LPAD), jnp.float32),
            pltpu.SemaphoreType.DMA,
            pltpu.SemaphoreType.DMA,
            pltpu.SemaphoreType.DMA,
            pltpu.SemaphoreType.DMA,
        ],
    )
    def gather_kernel(text_hbm, table_hbm, out_hbm, idx_v, rows_v, tile_v,
                      gsem0, gsem1, osem0, osem1):
        c = lax.axis_index("c")
        s_ax = lax.axis_index("s")
        wid = s_ax * _NC + c
        pair0 = wid * pairs_per_worker
        gsems = (gsem0, gsem1)
        osems = (osem0, osem1)
        iota16 = jax.lax.iota(jnp.int32, 16)

        pltpu.sync_copy(text_hbm.at[pl.ds(pair0 * _BLK, idx_per_worker)],
                        idx_v)

        def chunk_pos(g):
            p = pair0 + g * _PAIRS
            return p // n_bblk, p % n_bblk

        def gather_copies(g, b, fire):
            for j in range(_PAIRS):
                desc = pltpu.make_async_copy(
                    table_hbm.at[idx_v.at[pl.ds((g * _PAIRS + j) * _BLK,
                                                _BLK)]],
                    rows_v.at[b, pl.ds(j * _BLK, _BLK)],
                    gsems[b])
                desc.start() if fire else desc.wait()

        def out_copies(g, b, fire):
            s, bt0 = chunk_pos(g)
            for dt in range(n_dblk):
                desc = pltpu.make_async_copy(
                    tile_v.at[b, dt, pl.ds(0, tile_rows), pl.ds(0, _BLK)],
                    out_hbm.at[s, pl.ds(dt * n_bblk * 8 + bt0 * 8,
                                        tile_rows)],
                    osems[b])
                desc.start() if fire else desc.wait()

        dt_vecs = [(iota16 // 8) + 2 * cc for cc in range(dim // 16)]
        sub_vec = iota16 % 8

        def transpose_chunk(b):
            def t_body(t, carry):
                p = t // (_BLK // 8)
                lbase = (t % (_BLK // 8)) * 8
                row_vec = sub_vec + p * 8
                for rr in range(8):
                    lane = lbase + rr
                    r = p * _BLK + lane
                    lane_vec = jnp.full((16,), 0, jnp.int32) + lane
                    for cc in range(dim // 16):
                        vals = rows_v[b, r, pl.ds(cc * 16, 16)]
                        plsc.store_scatter(
                            tile_v.at[b],
                            [dt_vecs[cc], row_vec, lane_vec], vals)
                return carry

            lax.fori_loop(0, _CHUNK // 8, t_body, 0)

        for b in range(_NBUF):
            gather_copies(b, b, fire=True)

        def loop_body(t, carry):
            for b in range(_NBUF):
                g = t * _NBUF + b
                gather_copies(g, b, fire=False)

                @pl.when(g >= _NBUF)
                def _():
                    out_copies(g - _NBUF, b, fire=False)

                transpose_chunk(b)
                out_copies(g, b, fire=True)

                @pl.when(g + _NBUF < chunks_per_worker)
                def _():
                    gather_copies(g + _NBUF, b, fire=True)
            return carry

        lax.fori_loop(0, chunks_per_worker // _NBUF, loop_body, 0)

        for b in range(_NBUF):
            out_copies(chunks_per_worker - _NBUF + b, b, fire=False)

    return gather_kernel


def kernel(text, table):
    batch, seq = text.shape
    vocab, dim = table.shape
    text_flat = jnp.transpose(text).astype(jnp.int32).reshape(batch * seq)
    table_t = jnp.transpose(table)
    tail = vocab % _FCOLS
    tail_rows = table[vocab - tail:].reshape(tail // 2, 2 * dim)
    scratch = _build_table_format(vocab, dim)(table_t, tail_rows)
    table_rm = scratch.reshape(vocab, dim)
    out3 = _build(batch, seq, vocab, dim)(text_flat, table_rm)
    n_bblk = batch // _BLK
    n_dblk = dim // 8
    out6 = out3.reshape(seq, n_dblk, n_bblk, 8, _BLK)
    return jnp.transpose(out6, (2, 4, 0, 1, 3)).reshape(batch, seq, dim)


# speedup vs baseline: 1.6221x; 1.6221x over previous
import functools

import jax
import jax.numpy as jnp
from jax import lax
from jax.experimental import pallas as pl
from jax.experimental.pallas import tpu as pltpu
from jax.experimental.pallas import tpu_sc as plsc

_NC = 2
_NS = 16
_NW = _NC * _NS
_BLK = 128
_PAIRS = 2
_CHUNK = _PAIRS * _BLK
_NBUF = 2
_LPAD = _BLK + 1


@functools.cache
def _build(batch, seq, vocab, dim):
    n_bblk = batch // _BLK
    n_pairs = seq * n_bblk
    pairs_per_worker = n_pairs // _NW
    chunks_per_worker = pairs_per_worker // _PAIRS
    assert chunks_per_worker % _NBUF == 0
    idx_per_worker = pairs_per_worker * _BLK
    n_dblk = dim // 8
    tile_rows = _PAIRS * 8

    mesh = plsc.VectorSubcoreMesh(core_axis_name="c", subcore_axis_name="s")

    @functools.partial(
        pl.kernel,
        out_type=jax.ShapeDtypeStruct((seq, n_dblk * n_bblk * 8, _BLK),
                                      jnp.float32),
        mesh=mesh,
        compiler_params=pltpu.CompilerParams(
            use_tc_tiling_on_sc=False, needs_layout_passes=False),
        scratch_types=[
            pltpu.VMEM((idx_per_worker,), jnp.int32),
            pltpu.VMEM((_NBUF, _CHUNK, dim), jnp.float32),
            pltpu.VMEM((_NBUF, n_dblk, tile_rows, _LPAD), jnp.float32),
            pltpu.SemaphoreType.DMA,
            pltpu.SemaphoreType.DMA,
            pltpu.SemaphoreType.DMA,
            pltpu.SemaphoreType.DMA,
        ],
    )
    def gather_kernel(text_hbm, table_hbm, out_hbm, idx_v, rows_v, tile_v,
                      gsem0, gsem1, osem0, osem1):
        c = lax.axis_index("c")
        s_ax = lax.axis_index("s")
        wid = s_ax * _NC + c
        pair0 = wid * pairs_per_worker
        gsems = (gsem0, gsem1)
        osems = (osem0, osem1)
        iota16 = jax.lax.iota(jnp.int32, 16)

        pltpu.sync_copy(text_hbm.at[pl.ds(pair0 * _BLK, idx_per_worker)],
                        idx_v)

        def chunk_pos(g):
            p = pair0 + g * _PAIRS
            return p // n_bblk, p % n_bblk

        def gather_copies(g, b, fire):
            for j in range(_PAIRS):
                desc = pltpu.make_async_copy(
                    table_hbm.at[idx_v.at[pl.ds((g * _PAIRS + j) * _BLK,
                                                _BLK)]],
                    rows_v.at[b, pl.ds(j * _BLK, _BLK)],
                    gsems[b])
                desc.start() if fire else desc.wait()

        def out_copies(g, b, fire):
            s, bt0 = chunk_pos(g)
            for dt in range(n_dblk):
                desc = pltpu.make_async_copy(
                    tile_v.at[b, dt, pl.ds(0, tile_rows), pl.ds(0, _BLK)],
                    out_hbm.at[s, pl.ds(dt * n_bblk * 8 + bt0 * 8,
                                        tile_rows)],
                    osems[b])
                desc.start() if fire else desc.wait()

        dt_vecs = [(iota16 // 8) + 2 * cc for cc in range(dim // 16)]
        sub_vec = iota16 % 8

        def transpose_chunk(b):
            def t_body(t, carry):
                p = t // (_BLK // 8)
                lbase = (t % (_BLK // 8)) * 8
                row_vec = sub_vec + p * 8
                for rr in range(8):
                    lane = lbase + rr
                    r = p * _BLK + lane
                    lane_vec = jnp.full((16,), 0, jnp.int32) + lane
                    for cc in range(dim // 16):
                        vals = rows_v[b, r, pl.ds(cc * 16, 16)]
                        plsc.store_scatter(
                            tile_v.at[b],
                            [dt_vecs[cc], row_vec, lane_vec], vals)
                return carry

            lax.fori_loop(0, _CHUNK // 8, t_body, 0)

        for b in range(_NBUF):
            gather_copies(b, b, fire=True)

        def loop_body(t, carry):
            for b in range(_NBUF):
                g = t * _NBUF + b
                gather_copies(g, b, fire=False)

                @pl.when(g >= _NBUF)
                def _():
                    out_copies(g - _NBUF, b, fire=False)

                transpose_chunk(b)
                out_copies(g, b, fire=True)

                @pl.when(g + _NBUF < chunks_per_worker)
                def _():
                    gather_copies(g + _NBUF, b, fire=True)
            return carry

        lax.fori_loop(0, chunks_per_worker // _NBUF, loop_body, 0)

        for b in range(_NBUF):
            out_copies(chunks_per_worker - _NBUF + b, b, fire=False)

    return gather_kernel


def kernel(text, table):
    batch, seq = text.shape
    vocab, dim = table.shape
    text_flat = jnp.transpose(text).astype(jnp.int32).reshape(batch * seq)
    out3 = _build(batch, seq, vocab, dim)(text_flat, table)
    n_bblk = batch // _BLK
    n_dblk = dim // 8
    out6 = out3.reshape(seq, n_dblk, n_bblk, 8, _BLK)
    return jnp.transpose(out6, (2, 4, 0, 1, 3)).reshape(batch, seq, dim)
